# Initial kernel scaffold; baseline (speedup 1.0000x reference)
import jax
import jax.numpy as jnp
from jax.experimental import pallas as pl


def kernel(x, edge_index, fc1_W, fc1_b, bn1_g, bn1_b, Wroot, Wnbr, conv_b, fc2_W, fc2_b, bn2_g, bn2_b):
    raise NotImplementedError("write your pallas kernel here")


# speedup vs baseline: 4.4103x; 4.4103x over previous
import functools

import jax
import jax.numpy as jnp
from jax import lax
from jax.experimental import pallas as pl
from jax.experimental.pallas import tpu as pltpu
from jax.experimental.pallas import tpu_sc as plsc

_EPS = 1e-5
_Q = 4
_QW = 64


def _xstats_body(x_ref, g_ref, m_ref):
    i = pl.program_id(0)
    xb = x_ref[...]
    gg = lax.dot_general(xb, xb, (((0,), (0,)), ((), ())),
                         preferred_element_type=jnp.float32)
    ms = jnp.sum(xb, axis=0, keepdims=True)

    @pl.when(i == 0)
    def _():
        g_ref[...] = gg
        m_ref[...] = jnp.zeros_like(m_ref)
        m_ref[0:1, :] = ms

    @pl.when(i > 0)
    def _():
        g_ref[...] += gg
        m_ref[0:1, :] += ms


def _xstats(x, rows_per_blk):
    n, d = x.shape
    grid = n // rows_per_blk
    return pl.pallas_call(
        _xstats_body,
        grid=(grid,),
        in_specs=[pl.BlockSpec((rows_per_blk, d), lambda i: (i, 0))],
        out_specs=[pl.BlockSpec((d, d), lambda i: (0, 0)),
                   pl.BlockSpec((8, d), lambda i: (0, 0))],
        out_shape=[jax.ShapeDtypeStruct((d, d), jnp.float32),
                   jax.ShapeDtypeStruct((8, d), jnp.float32)],
    )(x)


def _h1_body(g_ref, m_ref, w1t_ref, g1_ref, be1_ref, x_ref, out_ref, *, n):
    w1t = w1t_ref[...]
    t = jnp.dot(g_ref[...], w1t, preferred_element_type=jnp.float32)
    ex2 = jnp.sum(w1t * t, axis=0, keepdims=True) / n
    mu0 = jnp.dot(m_ref[0:1, :], w1t,
                  preferred_element_type=jnp.float32) / n
    var = ex2 - mu0 * mu0
    a = g1_ref[...] * lax.rsqrt(var + _EPS)
    c = be1_ref[...] - a * mu0
    y = jnp.dot(x_ref[...], w1t, preferred_element_type=jnp.float32)
    h = a * y + c
    for q in range(_Q):
        out_ref[q] = h[:, q * _QW:(q + 1) * _QW]


def _h1(g, m, w1t, g1, be1, x, rows_per_blk):
    n, d = x.shape
    h = w1t.shape[1]
    grid = n // rows_per_blk
    body = functools.partial(_h1_body, n=n)
    return pl.pallas_call(
        body,
        grid=(grid,),
        in_specs=[pl.BlockSpec((d, d), lambda i: (0, 0)),
                  pl.BlockSpec((8, d), lambda i: (0, 0)),
                  pl.BlockSpec((d, h), lambda i: (0, 0)),
                  pl.BlockSpec((1, h), lambda i: (0, 0)),
                  pl.BlockSpec((1, h), lambda i: (0, 0)),
                  pl.BlockSpec((rows_per_blk, d), lambda i: (i, 0))],
        out_specs=pl.BlockSpec((_Q, rows_per_blk, _QW), lambda i: (0, i, 0)),
        out_shape=jax.ShapeDtypeStruct((_Q, n, _QW), jnp.float32),
    )(g, m, w1t, g1, be1, x)


def _sc_agg_body(h1_ref, src_ref, dst_ref, out_ref,
                 idxs_v, idxd_v, rows_v, zbuf_v, acc_sh, sem,
                 *, n, n_chunks, rows_per_tile_out, zero_copies):
    c = lax.axis_index("c")
    s = lax.axis_index("s")

    def _zrow(r, carry):
        for j in range(_QW // 16):
            zbuf_v[r, pl.ds(j * 16, 16)] = jnp.zeros((16,), jnp.float32)
        return carry
    lax.fori_loop(0, 128, _zrow, 0)

    pltpu.sync_copy(src_ref.at[c, s], idxs_v)
    pltpu.sync_copy(dst_ref.at[s], idxd_v)

    for p in range(2):
        if p == 1:
            def _bump(r, carry):
                for j in range(8):
                    v = idxs_v[r, pl.ds(j * 16, 16)]
                    idxs_v[r, pl.ds(j * 16, 16)] = v + n
                return carry
            lax.fori_loop(0, n_chunks, _bump, 0)

        def _zcopy(k, carry):
            pltpu.sync_copy(zbuf_v, acc_sh.at[pl.ds(s * (zero_copies * 128)
                                                    + k * 128, 128)])
            return carry
        lax.fori_loop(0, zero_copies, _zcopy, 0)
        plsc.subcore_barrier()

        def _edge_chunk(gi, carry):
            pltpu.async_copy(h1_ref.at[idxs_v.at[gi]], rows_v, sem).wait()
            pltpu.sync_copy(rows_v, acc_sh.at[idxd_v.at[gi]], add=True)
            return carry
        lax.fori_loop(0, n_chunks, _edge_chunk, 0)
        plsc.subcore_barrier()

        pltpu.sync_copy(
            acc_sh.at[pl.ds(s * rows_per_tile_out, rows_per_tile_out)],
            out_ref.at[2 * c + p, s])
        plsc.subcore_barrier()


def _sc_agg(h1_flat, src2, dst2, n, nsp):
    n_chunks = src2.shape[2]
    rows_per_tile_out = n // 16
    zero_copies = nsp // (16 * 128)
    body = functools.partial(_sc_agg_body, n=n, n_chunks=n_chunks,
                             rows_per_tile_out=rows_per_tile_out,
                             zero_copies=zero_copies)
    kern = pl.kernel(
        body,
        out_type=jax.ShapeDtypeStruct((_Q, 16, rows_per_tile_out, _QW),
                                      jnp.float32),
        mesh=plsc.VectorSubcoreMesh(core_axis_name="c", subcore_axis_name="s"),
        compiler_params=pltpu.CompilerParams(use_tc_tiling_on_sc=False),
        scratch_types=[
            pltpu.VMEM((n_chunks, 128), jnp.int32),
            pltpu.VMEM((n_chunks, 128), jnp.int32),
            pltpu.VMEM((128, _QW), jnp.float32),
            pltpu.VMEM((128, _QW), jnp.float32),
            pltpu.VMEM_SHARED((nsp, _QW), jnp.float32),
            pltpu.SemaphoreType.DMA,
        ],
    )
    return kern(h1_flat, src2, dst2)


def _h2_body(h1_ref, agg_ref, wr_ref, wn_ref, cb_ref, w2t_ref, b2_ref,
             y2_ref, st_ref):
    i = pl.program_id(0)
    h2 = cb_ref[...]
    for q in range(_Q):
        h2 = (h2
              + jnp.dot(h1_ref[q], wr_ref[q],
                        preferred_element_type=jnp.float32)
              + jnp.dot(agg_ref[q], wn_ref[q],
                        preferred_element_type=jnp.float32))
    y2 = jnp.dot(h2, w2t_ref[...], preferred_element_type=jnp.float32) \
        + b2_ref[...]
    y2_ref[...] = y2
    s1 = jnp.sum(y2, axis=0, keepdims=True)
    s2 = jnp.sum(y2 * y2, axis=0, keepdims=True)

    @pl.when(i == 0)
    def _():
        st_ref[...] = jnp.zeros_like(st_ref)
        st_ref[0:1, :] = s1
        st_ref[1:2, :] = s2

    @pl.when(i > 0)
    def _():
        st_ref[0:1, :] += s1
        st_ref[1:2, :] += s2


def _h2(h1s, aggs, wr, wn, cb, w2t, b2, rows_per_blk):
    _, n, _ = h1s.shape
    h2dim = wr.shape[2]
    d = w2t.shape[1]
    grid = n // rows_per_blk
    return pl.pallas_call(
        _h2_body,
        grid=(grid,),
        in_specs=[pl.BlockSpec((_Q, rows_per_blk, _QW), lambda i: (0, i, 0)),
                  pl.BlockSpec((_Q, rows_per_blk, _QW), lambda i: (0, i, 0)),
                  pl.BlockSpec((_Q, _QW, h2dim), lambda i: (0, 0, 0)),
                  pl.BlockSpec((_Q, _QW, h2dim), lambda i: (0, 0, 0)),
                  pl.BlockSpec((1, h2dim), lambda i: (0, 0)),
                  pl.BlockSpec((h2dim, d), lambda i: (0, 0)),
                  pl.BlockSpec((1, d), lambda i: (0, 0))],
        out_specs=[pl.BlockSpec((rows_per_blk, d), lambda i: (i, 0)),
                   pl.BlockSpec((8, d), lambda i: (0, 0))],
        out_shape=[jax.ShapeDtypeStruct((n, d), jnp.float32),
                   jax.ShapeDtypeStruct((8, d), jnp.float32)],
    )(h1s, aggs, wr, wn, cb, w2t, b2)


def _final_body(st_ref, g2_ref, be2_ref, y2_ref, x_ref, out_ref, *, n):
    mu = st_ref[0:1, :] / n
    ex2 = st_ref[1:2, :] / n
    var = ex2 - mu * mu
    a = g2_ref[...] * lax.rsqrt(var + _EPS)
    dd = be2_ref[...] - a * mu
    out_ref[...] = a * y2_ref[...] + dd + x_ref[...]


def _final(st, g2, be2, y2, x, rows_per_blk):
    n, d = x.shape
    grid = n // rows_per_blk
    body = functools.partial(_final_body, n=n)
    return pl.pallas_call(
        body,
        grid=(grid,),
        in_specs=[pl.BlockSpec((8, d), lambda i: (0, 0)),
                  pl.BlockSpec((1, d), lambda i: (0, 0)),
                  pl.BlockSpec((1, d), lambda i: (0, 0)),
                  pl.BlockSpec((rows_per_blk, d), lambda i: (i, 0)),
                  pl.BlockSpec((rows_per_blk, d), lambda i: (i, 0))],
        out_specs=pl.BlockSpec((rows_per_blk, d), lambda i: (i, 0)),
        out_shape=jax.ShapeDtypeStruct((n, d), jnp.float32),
    )(st, g2, be2, y2, x)


def kernel(x, edge_index, fc1_W, fc1_b, bn1_g, bn1_b, Wroot, Wnbr, conv_b,
           fc2_W, fc2_b, bn2_g, bn2_b):
    n, d = x.shape
    h = fc1_W.shape[0]
    e = edge_index.shape[1]
    rows_per_blk = 1000

    del fc1_b
    g, m = _xstats(x, rows_per_blk)
    w1t = fc1_W.T
    h1s = _h1(g, m, w1t, bn1_g.reshape(1, h), bn1_b.reshape(1, h), x,
              rows_per_blk)

    n_chunks = -(-e // (16 * 128))
    e_pad = n_chunks * 16 * 128
    nsp = -(-(n + 16) // 2048) * 2048
    pad = e_pad - e
    src = edge_index[0]
    dst = edge_index[1]
    fill = jnp.arange(pad, dtype=jnp.int32)
    src_p = jnp.concatenate([src, (fill * 97) % n])
    dst_p = jnp.concatenate([dst, n + fill % (nsp - n)])
    src2 = jnp.stack([src_p, src_p + 2 * n]).reshape(2, 16, n_chunks, 128)
    dst2 = dst_p.reshape(16, n_chunks, 128)

    h1_flat = h1s.reshape(_Q * n, _QW)
    agg4 = _sc_agg(h1_flat, src2, dst2, n, nsp)
    aggs = agg4.reshape(_Q, n, _QW)

    wr = Wroot.T.reshape(_Q, _QW, 2 * h)
    wn = Wnbr.T.reshape(_Q, _QW, 2 * h)
    w2t = fc2_W.T
    y2, st = _h2(h1s, aggs, wr, wn, conv_b.reshape(1, 2 * h), w2t,
                 fc2_b.reshape(1, d), rows_per_blk)
    return _final(st, bn2_g.reshape(1, d), bn2_b.reshape(1, d), y2, x,
                  rows_per_blk)


# speedup vs baseline: 5.9170x; 1.3416x over previous
import functools

import jax
import jax.numpy as jnp
from jax import lax
from jax.experimental import pallas as pl
from jax.experimental.pallas import tpu as pltpu
from jax.experimental.pallas import tpu_sc as plsc

_EPS = 1e-5
_Q = 4
_QW = 64


def _xstats_body(x_ref, g_ref, m_ref):
    i = pl.program_id(0)
    xb = x_ref[...]
    gg = lax.dot_general(xb, xb, (((0,), (0,)), ((), ())),
                         preferred_element_type=jnp.float32)
    ms = jnp.sum(xb, axis=0, keepdims=True)

    @pl.when(i == 0)
    def _():
        g_ref[...] = gg
        m_ref[...] = jnp.zeros_like(m_ref)
        m_ref[0:1, :] = ms

    @pl.when(i > 0)
    def _():
        g_ref[...] += gg
        m_ref[0:1, :] += ms


def _xstats(x, rows_per_blk):
    n, d = x.shape
    grid = n // rows_per_blk
    return pl.pallas_call(
        _xstats_body,
        grid=(grid,),
        in_specs=[pl.BlockSpec((rows_per_blk, d), lambda i: (i, 0))],
        out_specs=[pl.BlockSpec((d, d), lambda i: (0, 0)),
                   pl.BlockSpec((8, d), lambda i: (0, 0))],
        out_shape=[jax.ShapeDtypeStruct((d, d), jnp.float32),
                   jax.ShapeDtypeStruct((8, d), jnp.float32)],
    )(x)


def _h1_body(g_ref, m_ref, w1t_ref, g1_ref, be1_ref, x_ref, out_ref, *, n):
    w1t = w1t_ref[...]
    t = jnp.dot(g_ref[...], w1t, preferred_element_type=jnp.float32)
    ex2 = jnp.sum(w1t * t, axis=0, keepdims=True) / n
    mu0 = jnp.dot(m_ref[0:1, :], w1t,
                  preferred_element_type=jnp.float32) / n
    var = ex2 - mu0 * mu0
    a = g1_ref[...] * lax.rsqrt(var + _EPS)
    c = be1_ref[...] - a * mu0
    y = jnp.dot(x_ref[...], w1t, preferred_element_type=jnp.float32)
    h = a * y + c
    for q in range(_Q):
        out_ref[q] = h[:, q * _QW:(q + 1) * _QW]


def _h1(g, m, w1t, g1, be1, x, rows_per_blk):
    n, d = x.shape
    h = w1t.shape[1]
    grid = n // rows_per_blk
    body = functools.partial(_h1_body, n=n)
    return pl.pallas_call(
        body,
        grid=(grid,),
        in_specs=[pl.BlockSpec((d, d), lambda i: (0, 0)),
                  pl.BlockSpec((8, d), lambda i: (0, 0)),
                  pl.BlockSpec((d, h), lambda i: (0, 0)),
                  pl.BlockSpec((1, h), lambda i: (0, 0)),
                  pl.BlockSpec((1, h), lambda i: (0, 0)),
                  pl.BlockSpec((rows_per_blk, d), lambda i: (i, 0))],
        out_specs=pl.BlockSpec((_Q, rows_per_blk, _QW), lambda i: (0, i, 0)),
        out_shape=jax.ShapeDtypeStruct((_Q, n, _QW), jnp.float32),
    )(g, m, w1t, g1, be1, x)


_NBUF = 4


def _sc_agg_body(h1_ref, src_ref, dst_ref, out_ref,
                 idxs_v, idxd_v, rows_v, zbuf_v, acc_sh,
                 sem0, sem1, sem2, sem3,
                 *, n, n_chunks, rows_per_tile_out, zero_copies):
    sems = (sem0, sem1, sem2, sem3)
    c = lax.axis_index("c")
    s = lax.axis_index("s")

    def _zrow(r, carry):
        for j in range(_QW // 16):
            zbuf_v[r, pl.ds(j * 16, 16)] = jnp.zeros((16,), jnp.float32)
        return carry
    lax.fori_loop(0, 128, _zrow, 0)

    pltpu.sync_copy(src_ref.at[c, s], idxs_v)
    pltpu.sync_copy(dst_ref.at[s], idxd_v)

    for p in range(2):
        if p == 1:
            def _bump(r, carry):
                for j in range(8):
                    v = idxs_v[r, pl.ds(j * 16, 16)]
                    idxs_v[r, pl.ds(j * 16, 16)] = v + n
                return carry
            lax.fori_loop(0, n_chunks, _bump, 0)

        def _zcopy(k, carry):
            pltpu.sync_copy(zbuf_v, acc_sh.at[pl.ds(s * (zero_copies * 128)
                                                    + k * 128, 128)])
            return carry
        lax.fori_loop(0, zero_copies, _zcopy, 0)
        plsc.subcore_barrier()

        def _edge_group(gp, carry):
            base = gp * _NBUF
            cps = [pltpu.async_copy(h1_ref.at[idxs_v.at[base + j]],
                                    rows_v.at[j], sems[j])
                   for j in range(_NBUF)]
            for j in range(_NBUF):
                cps[j].wait()
                pltpu.sync_copy(rows_v.at[j], acc_sh.at[idxd_v.at[base + j]],
                                add=True)
            return carry
        lax.fori_loop(0, n_chunks // _NBUF, _edge_group, 0)
        plsc.subcore_barrier()

        pltpu.sync_copy(
            acc_sh.at[pl.ds(s * rows_per_tile_out, rows_per_tile_out)],
            out_ref.at[2 * c + p, s])
        plsc.subcore_barrier()


def _sc_agg(h1_flat, src2, dst2, n, nsp):
    n_chunks = src2.shape[2]
    rows_per_tile_out = n // 16
    zero_copies = nsp // (16 * 128)
    body = functools.partial(_sc_agg_body, n=n, n_chunks=n_chunks,
                             rows_per_tile_out=rows_per_tile_out,
                             zero_copies=zero_copies)
    kern = pl.kernel(
        body,
        out_type=jax.ShapeDtypeStruct((_Q, 16, rows_per_tile_out, _QW),
                                      jnp.float32),
        mesh=plsc.VectorSubcoreMesh(core_axis_name="c", subcore_axis_name="s"),
        compiler_params=pltpu.CompilerParams(use_tc_tiling_on_sc=False),
        scratch_types=[
            pltpu.VMEM((n_chunks, 128), jnp.int32),
            pltpu.VMEM((n_chunks, 128), jnp.int32),
            pltpu.VMEM((_NBUF, 128, _QW), jnp.float32),
            pltpu.VMEM((128, _QW), jnp.float32),
            pltpu.VMEM_SHARED((nsp, _QW), jnp.float32),
            pltpu.SemaphoreType.DMA,
            pltpu.SemaphoreType.DMA,
            pltpu.SemaphoreType.DMA,
            pltpu.SemaphoreType.DMA,
        ],
    )
    return kern(h1_flat, src2, dst2)


def _h2_body(h1_ref, agg_ref, wr_ref, wn_ref, cb_ref, w2t_ref, b2_ref,
             y2_ref, st_ref):
    i = pl.program_id(0)
    h2 = cb_ref[...]
    for q in range(_Q):
        h2 = (h2
              + jnp.dot(h1_ref[q], wr_ref[q],
                        preferred_element_type=jnp.float32)
              + jnp.dot(agg_ref[q], wn_ref[q],
                        preferred_element_type=jnp.float32))
    y2 = jnp.dot(h2, w2t_ref[...], preferred_element_type=jnp.float32) \
        + b2_ref[...]
    y2_ref[...] = y2
    s1 = jnp.sum(y2, axis=0, keepdims=True)
    s2 = jnp.sum(y2 * y2, axis=0, keepdims=True)

    @pl.when(i == 0)
    def _():
        st_ref[...] = jnp.zeros_like(st_ref)
        st_ref[0:1, :] = s1
        st_ref[1:2, :] = s2

    @pl.when(i > 0)
    def _():
        st_ref[0:1, :] += s1
        st_ref[1:2, :] += s2


def _h2(h1s, aggs, wr, wn, cb, w2t, b2, rows_per_blk):
    _, n, _ = h1s.shape
    h2dim = wr.shape[2]
    d = w2t.shape[1]
    grid = n // rows_per_blk
    return pl.pallas_call(
        _h2_body,
        grid=(grid,),
        in_specs=[pl.BlockSpec((_Q, rows_per_blk, _QW), lambda i: (0, i, 0)),
                  pl.BlockSpec((_Q, rows_per_blk, _QW), lambda i: (0, i, 0)),
                  pl.BlockSpec((_Q, _QW, h2dim), lambda i: (0, 0, 0)),
                  pl.BlockSpec((_Q, _QW, h2dim), lambda i: (0, 0, 0)),
                  pl.BlockSpec((1, h2dim), lambda i: (0, 0)),
                  pl.BlockSpec((h2dim, d), lambda i: (0, 0)),
                  pl.BlockSpec((1, d), lambda i: (0, 0))],
        out_specs=[pl.BlockSpec((rows_per_blk, d), lambda i: (i, 0)),
                   pl.BlockSpec((8, d), lambda i: (0, 0))],
        out_shape=[jax.ShapeDtypeStruct((n, d), jnp.float32),
                   jax.ShapeDtypeStruct((8, d), jnp.float32)],
    )(h1s, aggs, wr, wn, cb, w2t, b2)


def _final_body(st_ref, g2_ref, be2_ref, y2_ref, x_ref, out_ref, *, n):
    mu = st_ref[0:1, :] / n
    ex2 = st_ref[1:2, :] / n
    var = ex2 - mu * mu
    a = g2_ref[...] * lax.rsqrt(var + _EPS)
    dd = be2_ref[...] - a * mu
    out_ref[...] = a * y2_ref[...] + dd + x_ref[...]


def _final(st, g2, be2, y2, x, rows_per_blk):
    n, d = x.shape
    grid = n // rows_per_blk
    body = functools.partial(_final_body, n=n)
    return pl.pallas_call(
        body,
        grid=(grid,),
        in_specs=[pl.BlockSpec((8, d), lambda i: (0, 0)),
                  pl.BlockSpec((1, d), lambda i: (0, 0)),
                  pl.BlockSpec((1, d), lambda i: (0, 0)),
                  pl.BlockSpec((rows_per_blk, d), lambda i: (i, 0)),
                  pl.BlockSpec((rows_per_blk, d), lambda i: (i, 0))],
        out_specs=pl.BlockSpec((rows_per_blk, d), lambda i: (i, 0)),
        out_shape=jax.ShapeDtypeStruct((n, d), jnp.float32),
    )(st, g2, be2, y2, x)


def kernel(x, edge_index, fc1_W, fc1_b, bn1_g, bn1_b, Wroot, Wnbr, conv_b,
           fc2_W, fc2_b, bn2_g, bn2_b):
    n, d = x.shape
    h = fc1_W.shape[0]
    e = edge_index.shape[1]
    rows_per_blk = 1000

    del fc1_b
    g, m = _xstats(x, rows_per_blk)
    w1t = fc1_W.T
    h1s = _h1(g, m, w1t, bn1_g.reshape(1, h), bn1_b.reshape(1, h), x,
              rows_per_blk)

    n_chunks = -(-e // (16 * 128 * _NBUF)) * _NBUF
    e_pad = n_chunks * 16 * 128
    nsp = -(-(n + 16) // 2048) * 2048
    pad = e_pad - e
    src = edge_index[0]
    dst = edge_index[1]
    fill = jnp.arange(pad, dtype=jnp.int32)
    src_p = jnp.concatenate([src, (fill * 97) % n])
    dst_p = jnp.concatenate([dst, n + fill % (nsp - n)])
    src2 = jnp.stack([src_p, src_p + 2 * n]).reshape(2, 16, n_chunks, 128)
    dst2 = dst_p.reshape(16, n_chunks, 128)

    h1_flat = h1s.reshape(_Q * n, _QW)
    agg4 = _sc_agg(h1_flat, src2, dst2, n, nsp)
    aggs = agg4.reshape(_Q, n, _QW)

    wr = Wroot.T.reshape(_Q, _QW, 2 * h)
    wn = Wnbr.T.reshape(_Q, _QW, 2 * h)
    w2t = fc2_W.T
    y2, st = _h2(h1s, aggs, wr, wn, conv_b.reshape(1, 2 * h), w2t,
                 fc2_b.reshape(1, d), rows_per_blk)
    return _final(st, bn2_g.reshape(1, d), bn2_b.reshape(1, d), y2, x,
                  rows_per_blk)


# speedup vs baseline: 6.6940x; 1.1313x over previous
import functools

import jax
import jax.numpy as jnp
from jax import lax
from jax.experimental import pallas as pl
from jax.experimental.pallas import tpu as pltpu
from jax.experimental.pallas import tpu_sc as plsc

_EPS = 1e-5
_Q = 4
_QW = 64


def _xstats_body(x_ref, g_ref, m_ref):
    i = pl.program_id(0)
    xb = x_ref[...]
    gg = lax.dot_general(xb, xb, (((0,), (0,)), ((), ())),
                         preferred_element_type=jnp.float32)
    ms = jnp.sum(xb, axis=0, keepdims=True)

    @pl.when(i == 0)
    def _():
        g_ref[...] = gg
        m_ref[...] = jnp.zeros_like(m_ref)
        m_ref[0:1, :] = ms

    @pl.when(i > 0)
    def _():
        g_ref[...] += gg
        m_ref[0:1, :] += ms


def _xstats(x, rows_per_blk):
    n, d = x.shape
    grid = n // rows_per_blk
    return pl.pallas_call(
        _xstats_body,
        grid=(grid,),
        in_specs=[pl.BlockSpec((rows_per_blk, d), lambda i: (i, 0))],
        out_specs=[pl.BlockSpec((d, d), lambda i: (0, 0)),
                   pl.BlockSpec((8, d), lambda i: (0, 0))],
        out_shape=[jax.ShapeDtypeStruct((d, d), jnp.float32),
                   jax.ShapeDtypeStruct((8, d), jnp.float32)],
    )(x)


def _h1_body(g_ref, m_ref, w1t_ref, g1_ref, be1_ref, x_ref, out_ref, *, n):
    w1t = w1t_ref[...]
    t = jnp.dot(g_ref[...], w1t, preferred_element_type=jnp.float32)
    ex2 = jnp.sum(w1t * t, axis=0, keepdims=True) / n
    mu0 = jnp.dot(m_ref[0:1, :], w1t,
                  preferred_element_type=jnp.float32) / n
    var = ex2 - mu0 * mu0
    a = g1_ref[...] * lax.rsqrt(var + _EPS)
    c = be1_ref[...] - a * mu0
    y = jnp.dot(x_ref[...], w1t, preferred_element_type=jnp.float32)
    h = a * y + c
    for q in range(_Q):
        out_ref[q] = h[:, q * _QW:(q + 1) * _QW]


def _h1(g, m, w1t, g1, be1, x, rows_per_blk):
    n, d = x.shape
    h = w1t.shape[1]
    grid = n // rows_per_blk
    body = functools.partial(_h1_body, n=n)
    return pl.pallas_call(
        body,
        grid=(grid,),
        in_specs=[pl.BlockSpec((d, d), lambda i: (0, 0)),
                  pl.BlockSpec((8, d), lambda i: (0, 0)),
                  pl.BlockSpec((d, h), lambda i: (0, 0)),
                  pl.BlockSpec((1, h), lambda i: (0, 0)),
                  pl.BlockSpec((1, h), lambda i: (0, 0)),
                  pl.BlockSpec((rows_per_blk, d), lambda i: (i, 0))],
        out_specs=pl.BlockSpec((_Q, rows_per_blk, _QW), lambda i: (0, i, 0)),
        out_shape=jax.ShapeDtypeStruct((_Q, n, _QW), jnp.float32),
    )(g, m, w1t, g1, be1, x)


_NBUF = 8


def _sc_agg_body(h1_ref, src_ref, dst_ref, out_ref,
                 isrc_v, idst_v, rows_v, acc_sh,
                 *sems,
                 **kw):
    n_chunks = kw["n_chunks"]
    rows_per_tile_out = kw["rows_per_tile_out"]
    zero_copies = kw["zero_copies"]
    n_groups = n_chunks // _NBUF
    gsems = sems[:_NBUF]
    sem_is = sems[_NBUF:_NBUF + 2]
    sem_id = sems[_NBUF + 2:_NBUF + 4]
    c = lax.axis_index("c")
    s = lax.axis_index("s")

    def _start_idx(q, g, par):
        pltpu.async_copy(src_ref.at[q, s, pl.ds(g * _NBUF, _NBUF)],
                         isrc_v.at[par], sem_is[par])
        pltpu.async_copy(dst_ref.at[s, pl.ds(g * _NBUF, _NBUF)],
                         idst_v.at[par], sem_id[par])

    def _wait_idx(q, g, par):
        pltpu.make_async_copy(src_ref.at[q, s, pl.ds(g * _NBUF, _NBUF)],
                              isrc_v.at[par], sem_is[par]).wait()
        pltpu.make_async_copy(dst_ref.at[s, pl.ds(g * _NBUF, _NBUF)],
                              idst_v.at[par], sem_id[par]).wait()

    for p in range(2):
        q = 2 * c + p

        def _zrow(r, carry):
            for jj in range(_QW // 16):
                rows_v[0, r, pl.ds(jj * 16, 16)] = jnp.zeros((16,),
                                                             jnp.float32)
            return carry
        lax.fori_loop(0, 128, _zrow, 0)

        def _zcopy(k, carry):
            pltpu.sync_copy(rows_v.at[0],
                            acc_sh.at[pl.ds(s * (zero_copies * 128)
                                            + k * 128, 128)])
            return carry
        lax.fori_loop(0, zero_copies, _zcopy, 0)
        plsc.subcore_barrier()

        _start_idx(q, 0, 0)
        _start_idx(q, 1, 1)

        def _iter2(k, carry):
            for par in range(2):
                g = 2 * k + par
                _wait_idx(q, g, par)
                cps = [pltpu.async_copy(h1_ref.at[isrc_v.at[par, j]],
                                        rows_v.at[j], gsems[j])
                       for j in range(_NBUF)]
                for j in range(_NBUF):
                    cps[j].wait()
                    pltpu.sync_copy(rows_v.at[j],
                                    acc_sh.at[idst_v.at[par, j]], add=True)

                @pl.when(g + 2 < n_groups)
                def _():
                    _start_idx(q, g + 2, par)
            return carry
        lax.fori_loop(0, n_groups // 2, _iter2, 0)
        plsc.subcore_barrier()

        pltpu.sync_copy(
            acc_sh.at[pl.ds(s * rows_per_tile_out, rows_per_tile_out)],
            out_ref.at[q, s])
        plsc.subcore_barrier()


def _sc_agg(h1_flat, src4, dst2, n, nsp):
    n_chunks = src4.shape[2]
    rows_per_tile_out = n // 16
    zero_copies = nsp // (16 * 128)
    body = functools.partial(_sc_agg_body, n_chunks=n_chunks,
                             rows_per_tile_out=rows_per_tile_out,
                             zero_copies=zero_copies)
    kern = pl.kernel(
        body,
        out_type=jax.ShapeDtypeStruct((_Q, 16, rows_per_tile_out, _QW),
                                      jnp.float32),
        mesh=plsc.VectorSubcoreMesh(core_axis_name="c", subcore_axis_name="s"),
        compiler_params=pltpu.CompilerParams(use_tc_tiling_on_sc=False),
        scratch_types=[
            pltpu.VMEM((2, _NBUF, 128), jnp.int32),
            pltpu.VMEM((2, _NBUF, 128), jnp.int32),
            pltpu.VMEM((_NBUF, 128, _QW), jnp.float32),
            pltpu.VMEM_SHARED((nsp, _QW), jnp.float32),
        ] + [pltpu.SemaphoreType.DMA] * (_NBUF + 4),
    )
    return kern(h1_flat, src4, dst2)


def _h2_body(h1_ref, agg_ref, wr_ref, wn_ref, cb_ref, w2t_ref, b2_ref,
             y2_ref, st_ref):
    i = pl.program_id(0)
    h2 = cb_ref[...]
    for q in range(_Q):
        h2 = (h2
              + jnp.dot(h1_ref[q], wr_ref[q],
                        preferred_element_type=jnp.float32)
              + jnp.dot(agg_ref[q], wn_ref[q],
                        preferred_element_type=jnp.float32))
    y2 = jnp.dot(h2, w2t_ref[...], preferred_element_type=jnp.float32) \
        + b2_ref[...]
    y2_ref[...] = y2
    s1 = jnp.sum(y2, axis=0, keepdims=True)
    s2 = jnp.sum(y2 * y2, axis=0, keepdims=True)

    @pl.when(i == 0)
    def _():
        st_ref[...] = jnp.zeros_like(st_ref)
        st_ref[0:1, :] = s1
        st_ref[1:2, :] = s2

    @pl.when(i > 0)
    def _():
        st_ref[0:1, :] += s1
        st_ref[1:2, :] += s2


def _h2(h1s, aggs, wr, wn, cb, w2t, b2, rows_per_blk):
    _, n, _ = h1s.shape
    h2dim = wr.shape[2]
    d = w2t.shape[1]
    grid = n // rows_per_blk
    return pl.pallas_call(
        _h2_body,
        grid=(grid,),
        in_specs=[pl.BlockSpec((_Q, rows_per_blk, _QW), lambda i: (0, i, 0)),
                  pl.BlockSpec((_Q, rows_per_blk, _QW), lambda i: (0, i, 0)),
                  pl.BlockSpec((_Q, _QW, h2dim), lambda i: (0, 0, 0)),
                  pl.BlockSpec((_Q, _QW, h2dim), lambda i: (0, 0, 0)),
                  pl.BlockSpec((1, h2dim), lambda i: (0, 0)),
                  pl.BlockSpec((h2dim, d), lambda i: (0, 0)),
                  pl.BlockSpec((1, d), lambda i: (0, 0))],
        out_specs=[pl.BlockSpec((rows_per_blk, d), lambda i: (i, 0)),
                   pl.BlockSpec((8, d), lambda i: (0, 0))],
        out_shape=[jax.ShapeDtypeStruct((n, d), jnp.float32),
                   jax.ShapeDtypeStruct((8, d), jnp.float32)],
    )(h1s, aggs, wr, wn, cb, w2t, b2)


def _final_body(st_ref, g2_ref, be2_ref, y2_ref, x_ref, out_ref, *, n):
    mu = st_ref[0:1, :] / n
    ex2 = st_ref[1:2, :] / n
    var = ex2 - mu * mu
    a = g2_ref[...] * lax.rsqrt(var + _EPS)
    dd = be2_ref[...] - a * mu
    out_ref[...] = a * y2_ref[...] + dd + x_ref[...]


def _final(st, g2, be2, y2, x, rows_per_blk):
    n, d = x.shape
    grid = n // rows_per_blk
    body = functools.partial(_final_body, n=n)
    return pl.pallas_call(
        body,
        grid=(grid,),
        in_specs=[pl.BlockSpec((8, d), lambda i: (0, 0)),
                  pl.BlockSpec((1, d), lambda i: (0, 0)),
                  pl.BlockSpec((1, d), lambda i: (0, 0)),
                  pl.BlockSpec((rows_per_blk, d), lambda i: (i, 0)),
                  pl.BlockSpec((rows_per_blk, d), lambda i: (i, 0))],
        out_specs=pl.BlockSpec((rows_per_blk, d), lambda i: (i, 0)),
        out_shape=jax.ShapeDtypeStruct((n, d), jnp.float32),
    )(st, g2, be2, y2, x)


def kernel(x, edge_index, fc1_W, fc1_b, bn1_g, bn1_b, Wroot, Wnbr, conv_b,
           fc2_W, fc2_b, bn2_g, bn2_b):
    n, d = x.shape
    h = fc1_W.shape[0]
    e = edge_index.shape[1]
    rows_per_blk = 1000

    del fc1_b
    g, m = _xstats(x, rows_per_blk)
    w1t = fc1_W.T
    h1s = _h1(g, m, w1t, bn1_g.reshape(1, h), bn1_b.reshape(1, h), x,
              rows_per_blk)

    n_chunks = -(-e // (16 * 128 * 2 * _NBUF)) * 2 * _NBUF
    e_pad = n_chunks * 16 * 128
    nsp = -(-(n + 16) // 2048) * 2048
    pad = e_pad - e
    src = edge_index[0]
    dst = edge_index[1]
    fill = jnp.arange(pad, dtype=jnp.int32)
    src_p = jnp.concatenate([src, (fill * 97) % n])
    dst_p = jnp.concatenate([dst, n + fill % (nsp - n)])
    qoff = jnp.arange(_Q, dtype=jnp.int32)[:, None] * n
    src4 = (src_p[None, :] + qoff).reshape(_Q, 16, n_chunks, 128)
    dst2 = dst_p.reshape(16, n_chunks, 128)

    h1_flat = h1s.reshape(_Q * n, _QW)
    agg4 = _sc_agg(h1_flat, src4, dst2, n, nsp)
    aggs = agg4.reshape(_Q, n, _QW)

    wr = Wroot.T.reshape(_Q, _QW, 2 * h)
    wn = Wnbr.T.reshape(_Q, _QW, 2 * h)
    w2t = fc2_W.T
    y2, st = _h2(h1s, aggs, wr, wn, conv_b.reshape(1, 2 * h), w2t,
                 fc2_b.reshape(1, d), rows_per_blk)
    return _final(st, bn2_g.reshape(1, d), bn2_b.reshape(1, d), y2, x,
                  rows_per_blk)


# speedup vs baseline: 6.8349x; 1.0211x over previous
import functools

import jax
import jax.numpy as jnp
from jax import lax
from jax.experimental import pallas as pl
from jax.experimental.pallas import tpu as pltpu
from jax.experimental.pallas import tpu_sc as plsc

_EPS = 1e-5
_Q = 4
_QW = 64


def _xstats_body(x_ref, g_ref, m_ref):
    i = pl.program_id(0)
    xb = x_ref[...]
    gg = lax.dot_general(xb, xb, (((0,), (0,)), ((), ())),
                         preferred_element_type=jnp.float32)
    ms = jnp.sum(xb, axis=0, keepdims=True)

    @pl.when(i == 0)
    def _():
        g_ref[...] = gg
        m_ref[...] = jnp.zeros_like(m_ref)
        m_ref[0:1, :] = ms

    @pl.when(i > 0)
    def _():
        g_ref[...] += gg
        m_ref[0:1, :] += ms


def _xstats(x, rows_per_blk):
    n, d = x.shape
    grid = n // rows_per_blk
    return pl.pallas_call(
        _xstats_body,
        grid=(grid,),
        in_specs=[pl.BlockSpec((rows_per_blk, d), lambda i: (i, 0))],
        out_specs=[pl.BlockSpec((d, d), lambda i: (0, 0)),
                   pl.BlockSpec((8, d), lambda i: (0, 0))],
        out_shape=[jax.ShapeDtypeStruct((d, d), jnp.float32),
                   jax.ShapeDtypeStruct((8, d), jnp.float32)],
    )(x)


def _h1_body(g_ref, m_ref, w1t_ref, g1_ref, be1_ref, x_ref, out_ref, ac_ref,
             *, n):
    @pl.when(pl.program_id(0) == 0)
    def _():
        w1t = w1t_ref[...]
        t = jnp.dot(g_ref[...], w1t, preferred_element_type=jnp.float32)
        ex2 = jnp.sum(w1t * t, axis=0, keepdims=True) / n
        mu0 = jnp.dot(m_ref[0:1, :], w1t,
                      preferred_element_type=jnp.float32) / n
        var = ex2 - mu0 * mu0
        a = g1_ref[...] * lax.rsqrt(var + _EPS)
        ac_ref[0:1, :] = a
        ac_ref[1:2, :] = be1_ref[...] - a * mu0
    y = jnp.dot(x_ref[...], w1t_ref[...], preferred_element_type=jnp.float32)
    h = ac_ref[0:1, :] * y + ac_ref[1:2, :]
    for q in range(_Q):
        out_ref[q] = h[:, q * _QW:(q + 1) * _QW]


def _h1(g, m, w1t, g1, be1, x, rows_per_blk):
    n, d = x.shape
    h = w1t.shape[1]
    grid = n // rows_per_blk
    body = functools.partial(_h1_body, n=n)
    return pl.pallas_call(
        body,
        grid=(grid,),
        in_specs=[pl.BlockSpec((d, d), lambda i: (0, 0)),
                  pl.BlockSpec((8, d), lambda i: (0, 0)),
                  pl.BlockSpec((d, h), lambda i: (0, 0)),
                  pl.BlockSpec((1, h), lambda i: (0, 0)),
                  pl.BlockSpec((1, h), lambda i: (0, 0)),
                  pl.BlockSpec((rows_per_blk, d), lambda i: (i, 0))],
        out_specs=pl.BlockSpec((_Q, rows_per_blk, _QW), lambda i: (0, i, 0)),
        out_shape=jax.ShapeDtypeStruct((_Q, n, _QW), jnp.float32),
        scratch_shapes=[pltpu.VMEM((8, h), jnp.float32)],
    )(g, m, w1t, g1, be1, x)


_NBUF = 8


def _sc_agg_body(h1_ref, src_ref, dst_ref, out_ref,
                 isrc_v, idst_v, rows_v, acc_sh,
                 *sems,
                 **kw):
    n_chunks = kw["n_chunks"]
    rows_per_tile_out = kw["rows_per_tile_out"]
    zero_copies = kw["zero_copies"]
    n_groups = n_chunks // _NBUF
    gsems = sems[:_NBUF]
    sem_is = sems[_NBUF:_NBUF + 2]
    sem_id = sems[_NBUF + 2:_NBUF + 4]
    c = lax.axis_index("c")
    s = lax.axis_index("s")

    def _start_idx(q, g, par):
        pltpu.async_copy(src_ref.at[q, s, pl.ds(g * _NBUF, _NBUF)],
                         isrc_v.at[par], sem_is[par])
        pltpu.async_copy(dst_ref.at[s, pl.ds(g * _NBUF, _NBUF)],
                         idst_v.at[par], sem_id[par])

    def _wait_idx(q, g, par):
        pltpu.make_async_copy(src_ref.at[q, s, pl.ds(g * _NBUF, _NBUF)],
                              isrc_v.at[par], sem_is[par]).wait()
        pltpu.make_async_copy(dst_ref.at[s, pl.ds(g * _NBUF, _NBUF)],
                              idst_v.at[par], sem_id[par]).wait()

    for p in range(2):
        q = 2 * c + p

        def _zrow(r, carry):
            for jj in range(_QW // 16):
                rows_v[0, r, pl.ds(jj * 16, 16)] = jnp.zeros((16,),
                                                             jnp.float32)
            return carry
        lax.fori_loop(0, 128, _zrow, 0)

        def _zcopy(k, carry):
            pltpu.sync_copy(rows_v.at[0],
                            acc_sh.at[pl.ds(s * (zero_copies * 128)
                                            + k * 128, 128)])
            return carry
        lax.fori_loop(0, zero_copies, _zcopy, 0)
        plsc.subcore_barrier()

        _start_idx(q, 0, 0)
        _start_idx(q, 1, 1)

        def _iter2(k, carry):
            for par in range(2):
                g = 2 * k + par
                _wait_idx(q, g, par)
                cps = [pltpu.async_copy(h1_ref.at[isrc_v.at[par, j]],
                                        rows_v.at[j], gsems[j])
                       for j in range(_NBUF)]
                for j in range(_NBUF):
                    cps[j].wait()
                    pltpu.sync_copy(rows_v.at[j],
                                    acc_sh.at[idst_v.at[par, j]], add=True)

                @pl.when(g + 2 < n_groups)
                def _():
                    _start_idx(q, g + 2, par)
            return carry
        lax.fori_loop(0, n_groups // 2, _iter2, 0)
        plsc.subcore_barrier()

        pltpu.sync_copy(
            acc_sh.at[pl.ds(s * rows_per_tile_out, rows_per_tile_out)],
            out_ref.at[q, s])
        plsc.subcore_barrier()


def _sc_agg(h1_flat, src4, dst2, n, nsp):
    n_chunks = src4.shape[2]
    rows_per_tile_out = n // 16
    zero_copies = nsp // (16 * 128)
    body = functools.partial(_sc_agg_body, n_chunks=n_chunks,
                             rows_per_tile_out=rows_per_tile_out,
                             zero_copies=zero_copies)
    kern = pl.kernel(
        body,
        out_type=jax.ShapeDtypeStruct((_Q, 16, rows_per_tile_out, _QW),
                                      jnp.float32),
        mesh=plsc.VectorSubcoreMesh(core_axis_name="c", subcore_axis_name="s"),
        compiler_params=pltpu.CompilerParams(use_tc_tiling_on_sc=False),
        scratch_types=[
            pltpu.VMEM((2, _NBUF, 128), jnp.int32),
            pltpu.VMEM((2, _NBUF, 128), jnp.int32),
            pltpu.VMEM((_NBUF, 128, _QW), jnp.float32),
            pltpu.VMEM_SHARED((nsp, _QW), jnp.float32),
        ] + [pltpu.SemaphoreType.DMA] * (_NBUF + 4),
    )
    return kern(h1_flat, src4, dst2)


def _h2_body(h1_ref, agg_ref, wr_ref, wn_ref, cb_ref, w2t_ref, b2_ref,
             y2_ref, st_ref):
    i = pl.program_id(0)
    h2 = cb_ref[...]
    for q in range(_Q):
        h2 = (h2
              + jnp.dot(h1_ref[q], wr_ref[q],
                        preferred_element_type=jnp.float32)
              + jnp.dot(agg_ref[q], wn_ref[q],
                        preferred_element_type=jnp.float32))
    y2 = jnp.dot(h2, w2t_ref[...], preferred_element_type=jnp.float32) \
        + b2_ref[...]
    y2_ref[...] = y2
    s1 = jnp.sum(y2, axis=0, keepdims=True)
    s2 = jnp.sum(y2 * y2, axis=0, keepdims=True)

    @pl.when(i == 0)
    def _():
        st_ref[...] = jnp.zeros_like(st_ref)
        st_ref[0:1, :] = s1
        st_ref[1:2, :] = s2

    @pl.when(i > 0)
    def _():
        st_ref[0:1, :] += s1
        st_ref[1:2, :] += s2


def _h2(h1s, aggs, wr, wn, cb, w2t, b2, rows_per_blk):
    _, n, _ = h1s.shape
    h2dim = wr.shape[2]
    d = w2t.shape[1]
    grid = n // rows_per_blk
    return pl.pallas_call(
        _h2_body,
        grid=(grid,),
        in_specs=[pl.BlockSpec((_Q, rows_per_blk, _QW), lambda i: (0, i, 0)),
                  pl.BlockSpec((_Q, rows_per_blk, _QW), lambda i: (0, i, 0)),
                  pl.BlockSpec((_Q, _QW, h2dim), lambda i: (0, 0, 0)),
                  pl.BlockSpec((_Q, _QW, h2dim), lambda i: (0, 0, 0)),
                  pl.BlockSpec((1, h2dim), lambda i: (0, 0)),
                  pl.BlockSpec((h2dim, d), lambda i: (0, 0)),
                  pl.BlockSpec((1, d), lambda i: (0, 0))],
        out_specs=[pl.BlockSpec((rows_per_blk, d), lambda i: (i, 0)),
                   pl.BlockSpec((8, d), lambda i: (0, 0))],
        out_shape=[jax.ShapeDtypeStruct((n, d), jnp.float32),
                   jax.ShapeDtypeStruct((8, d), jnp.float32)],
    )(h1s, aggs, wr, wn, cb, w2t, b2)


def _final_body(st_ref, g2_ref, be2_ref, y2_ref, x_ref, out_ref, *, n):
    mu = st_ref[0:1, :] / n
    ex2 = st_ref[1:2, :] / n
    var = ex2 - mu * mu
    a = g2_ref[...] * lax.rsqrt(var + _EPS)
    dd = be2_ref[...] - a * mu
    out_ref[...] = a * y2_ref[...] + dd + x_ref[...]


def _final(st, g2, be2, y2, x, rows_per_blk):
    n, d = x.shape
    grid = n // rows_per_blk
    body = functools.partial(_final_body, n=n)
    return pl.pallas_call(
        body,
        grid=(grid,),
        in_specs=[pl.BlockSpec((8, d), lambda i: (0, 0)),
                  pl.BlockSpec((1, d), lambda i: (0, 0)),
                  pl.BlockSpec((1, d), lambda i: (0, 0)),
                  pl.BlockSpec((rows_per_blk, d), lambda i: (i, 0)),
                  pl.BlockSpec((rows_per_blk, d), lambda i: (i, 0))],
        out_specs=pl.BlockSpec((rows_per_blk, d), lambda i: (i, 0)),
        out_shape=jax.ShapeDtypeStruct((n, d), jnp.float32),
    )(st, g2, be2, y2, x)


def kernel(x, edge_index, fc1_W, fc1_b, bn1_g, bn1_b, Wroot, Wnbr, conv_b,
           fc2_W, fc2_b, bn2_g, bn2_b):
    n, d = x.shape
    h = fc1_W.shape[0]
    e = edge_index.shape[1]
    rows_per_blk = 2000

    del fc1_b
    g, m = _xstats(x, rows_per_blk)
    w1t = fc1_W.T
    h1s = _h1(g, m, w1t, bn1_g.reshape(1, h), bn1_b.reshape(1, h), x,
              rows_per_blk)

    n_chunks = -(-e // (16 * 128 * 2 * _NBUF)) * 2 * _NBUF
    e_pad = n_chunks * 16 * 128
    nsp = -(-(n + 16) // 2048) * 2048
    pad = e_pad - e
    src = edge_index[0]
    dst = edge_index[1]
    fill = jnp.arange(pad, dtype=jnp.int32)
    src_p = jnp.concatenate([src, (fill * 97) % n])
    dst_p = jnp.concatenate([dst, n + fill % (nsp - n)])
    qoff = jnp.arange(_Q, dtype=jnp.int32)[:, None] * n
    src4 = (src_p[None, :] + qoff).reshape(_Q, 16, n_chunks, 128)
    dst2 = dst_p.reshape(16, n_chunks, 128)

    h1_flat = h1s.reshape(_Q * n, _QW)
    agg4 = _sc_agg(h1_flat, src4, dst2, n, nsp)
    aggs = agg4.reshape(_Q, n, _QW)

    wr = Wroot.T.reshape(_Q, _QW, 2 * h)
    wn = Wnbr.T.reshape(_Q, _QW, 2 * h)
    w2t = fc2_W.T
    y2, st = _h2(h1s, aggs, wr, wn, conv_b.reshape(1, 2 * h), w2t,
                 fc2_b.reshape(1, d), rows_per_blk)
    return _final(st, bn2_g.reshape(1, d), bn2_b.reshape(1, d), y2, x,
                  rows_per_blk)


# speedup vs baseline: 7.1402x; 1.0447x over previous
import functools

import jax
import jax.numpy as jnp
from jax import lax
from jax.experimental import pallas as pl
from jax.experimental.pallas import tpu as pltpu
from jax.experimental.pallas import tpu_sc as plsc

_EPS = 1e-5
_Q = 4
_QW = 64


def _xstats_body(x_ref, g_ref, m_ref):
    i = pl.program_id(0)
    xb = x_ref[...]
    gg = lax.dot_general(xb, xb, (((0,), (0,)), ((), ())),
                         preferred_element_type=jnp.float32)
    ms = jnp.sum(xb, axis=0, keepdims=True)

    @pl.when(i == 0)
    def _():
        g_ref[...] = gg
        m_ref[...] = jnp.zeros_like(m_ref)
        m_ref[0:1, :] = ms

    @pl.when(i > 0)
    def _():
        g_ref[...] += gg
        m_ref[0:1, :] += ms


def _xstats(x, rows_per_blk):
    n, d = x.shape
    grid = n // rows_per_blk
    return pl.pallas_call(
        _xstats_body,
        grid=(grid,),
        in_specs=[pl.BlockSpec((rows_per_blk, d), lambda i: (i, 0))],
        out_specs=[pl.BlockSpec((d, d), lambda i: (0, 0)),
                   pl.BlockSpec((8, d), lambda i: (0, 0))],
        out_shape=[jax.ShapeDtypeStruct((d, d), jnp.float32),
                   jax.ShapeDtypeStruct((8, d), jnp.float32)],
    )(x)


def _h1_body(g_ref, m_ref, w1t_ref, g1_ref, be1_ref, x_ref, out_ref, ac_ref,
             *, n):
    @pl.when(pl.program_id(0) == 0)
    def _():
        w1t = w1t_ref[...]
        t = jnp.dot(g_ref[...], w1t, preferred_element_type=jnp.float32)
        ex2 = jnp.sum(w1t * t, axis=0, keepdims=True) / n
        mu0 = jnp.dot(m_ref[0:1, :], w1t,
                      preferred_element_type=jnp.float32) / n
        var = ex2 - mu0 * mu0
        a = g1_ref[...] * lax.rsqrt(var + _EPS)
        ac_ref[0:1, :] = a
        ac_ref[1:2, :] = be1_ref[...] - a * mu0
    y = jnp.dot(x_ref[...], w1t_ref[...], preferred_element_type=jnp.float32)
    h = ac_ref[0:1, :] * y + ac_ref[1:2, :]
    for q in range(_Q):
        out_ref[q] = h[:, q * _QW:(q + 1) * _QW]


def _h1(g, m, w1t, g1, be1, x, rows_per_blk):
    n, d = x.shape
    h = w1t.shape[1]
    grid = n // rows_per_blk
    body = functools.partial(_h1_body, n=n)
    return pl.pallas_call(
        body,
        grid=(grid,),
        in_specs=[pl.BlockSpec((d, d), lambda i: (0, 0)),
                  pl.BlockSpec((8, d), lambda i: (0, 0)),
                  pl.BlockSpec((d, h), lambda i: (0, 0)),
                  pl.BlockSpec((1, h), lambda i: (0, 0)),
                  pl.BlockSpec((1, h), lambda i: (0, 0)),
                  pl.BlockSpec((rows_per_blk, d), lambda i: (i, 0))],
        out_specs=pl.BlockSpec((_Q, rows_per_blk, _QW), lambda i: (0, i, 0)),
        out_shape=jax.ShapeDtypeStruct((_Q, n, _QW), jnp.float32),
        scratch_shapes=[pltpu.VMEM((8, h), jnp.float32)],
    )(g, m, w1t, g1, be1, x)


_NBUF = 10


def _sc_agg_body(h1_ref, src_ref, dst_ref, out_ref,
                 isrc_v, idst_v, rows_v, acc_sh,
                 *sems,
                 **kw):
    n_chunks = kw["n_chunks"]
    rows_per_tile_out = kw["rows_per_tile_out"]
    zero_copies = kw["zero_copies"]
    n_groups = n_chunks // _NBUF
    gsems = sems[:_NBUF]
    sem_is = sems[_NBUF:_NBUF + 2]
    sem_id = sems[_NBUF + 2:_NBUF + 4]
    c = lax.axis_index("c")
    s = lax.axis_index("s")

    def _start_idx(q, g, par):
        pltpu.async_copy(src_ref.at[q, s, pl.ds(g * _NBUF, _NBUF)],
                         isrc_v.at[par], sem_is[par])
        pltpu.async_copy(dst_ref.at[s, pl.ds(g * _NBUF, _NBUF)],
                         idst_v.at[par], sem_id[par])

    def _wait_idx(q, g, par):
        pltpu.make_async_copy(src_ref.at[q, s, pl.ds(g * _NBUF, _NBUF)],
                              isrc_v.at[par], sem_is[par]).wait()
        pltpu.make_async_copy(dst_ref.at[s, pl.ds(g * _NBUF, _NBUF)],
                              idst_v.at[par], sem_id[par]).wait()

    for p in range(2):
        q = 2 * c + p

        def _zrow(r, carry):
            for jj in range(_QW // 16):
                rows_v[0, r, pl.ds(jj * 16, 16)] = jnp.zeros((16,),
                                                             jnp.float32)
            return carry
        lax.fori_loop(0, 128, _zrow, 0)

        def _zcopy(k, carry):
            pltpu.sync_copy(rows_v.at[0],
                            acc_sh.at[pl.ds(s * (zero_copies * 128)
                                            + k * 128, 128)])
            return carry
        lax.fori_loop(0, zero_copies, _zcopy, 0)
        plsc.subcore_barrier()

        _start_idx(q, 0, 0)
        _start_idx(q, 1, 1)

        def _iter2(k, carry):
            for par in range(2):
                g = 2 * k + par
                _wait_idx(q, g, par)
                cps = [pltpu.async_copy(h1_ref.at[isrc_v.at[par, j]],
                                        rows_v.at[j], gsems[j])
                       for j in range(_NBUF)]
                for j in range(_NBUF):
                    cps[j].wait()
                    pltpu.sync_copy(rows_v.at[j],
                                    acc_sh.at[idst_v.at[par, j]], add=True)

                @pl.when(g + 2 < n_groups)
                def _():
                    _start_idx(q, g + 2, par)
            return carry
        lax.fori_loop(0, n_groups // 2, _iter2, 0)
        plsc.subcore_barrier()

        pltpu.sync_copy(
            acc_sh.at[pl.ds(s * rows_per_tile_out, rows_per_tile_out)],
            out_ref.at[q, s])
        plsc.subcore_barrier()


def _sc_agg(h1_flat, src4, dst2, n, nsp):
    n_chunks = src4.shape[2]
    rows_per_tile_out = n // 16
    zero_copies = nsp // (16 * 128)
    body = functools.partial(_sc_agg_body, n_chunks=n_chunks,
                             rows_per_tile_out=rows_per_tile_out,
                             zero_copies=zero_copies)
    kern = pl.kernel(
        body,
        out_type=jax.ShapeDtypeStruct((_Q, 16, rows_per_tile_out, _QW),
                                      jnp.float32),
        mesh=plsc.VectorSubcoreMesh(core_axis_name="c", subcore_axis_name="s"),
        compiler_params=pltpu.CompilerParams(use_tc_tiling_on_sc=False),
        scratch_types=[
            pltpu.VMEM((2, _NBUF, 128), jnp.int32),
            pltpu.VMEM((2, _NBUF, 128), jnp.int32),
            pltpu.VMEM((_NBUF, 128, _QW), jnp.float32),
            pltpu.VMEM_SHARED((nsp, _QW), jnp.float32),
        ] + [pltpu.SemaphoreType.DMA] * (_NBUF + 4),
    )
    return kern(h1_flat, src4, dst2)


def _h2_body(h1_ref, agg_ref, wr_ref, wn_ref, cb_ref, w2t_ref, b2_ref,
             y2_ref, st_ref):
    i = pl.program_id(0)
    h2 = cb_ref[...]
    for q in range(_Q):
        h2 = (h2
              + jnp.dot(h1_ref[q], wr_ref[q],
                        preferred_element_type=jnp.float32)
              + jnp.dot(agg_ref[q], wn_ref[q],
                        preferred_element_type=jnp.float32))
    y2 = jnp.dot(h2, w2t_ref[...], preferred_element_type=jnp.float32) \
        + b2_ref[...]
    y2_ref[...] = y2
    s1 = jnp.sum(y2, axis=0, keepdims=True)
    s2 = jnp.sum(y2 * y2, axis=0, keepdims=True)

    @pl.when(i == 0)
    def _():
        st_ref[...] = jnp.zeros_like(st_ref)
        st_ref[0:1, :] = s1
        st_ref[1:2, :] = s2

    @pl.when(i > 0)
    def _():
        st_ref[0:1, :] += s1
        st_ref[1:2, :] += s2


def _h2(h1s, aggs, wr, wn, cb, w2t, b2, rows_per_blk):
    _, n, _ = h1s.shape
    h2dim = wr.shape[2]
    d = w2t.shape[1]
    grid = n // rows_per_blk
    return pl.pallas_call(
        _h2_body,
        grid=(grid,),
        in_specs=[pl.BlockSpec((_Q, rows_per_blk, _QW), lambda i: (0, i, 0)),
                  pl.BlockSpec((_Q, rows_per_blk, _QW), lambda i: (0, i, 0)),
                  pl.BlockSpec((_Q, _QW, h2dim), lambda i: (0, 0, 0)),
                  pl.BlockSpec((_Q, _QW, h2dim), lambda i: (0, 0, 0)),
                  pl.BlockSpec((1, h2dim), lambda i: (0, 0)),
                  pl.BlockSpec((h2dim, d), lambda i: (0, 0)),
                  pl.BlockSpec((1, d), lambda i: (0, 0))],
        out_specs=[pl.BlockSpec((rows_per_blk, d), lambda i: (i, 0)),
                   pl.BlockSpec((8, d), lambda i: (0, 0))],
        out_shape=[jax.ShapeDtypeStruct((n, d), jnp.float32),
                   jax.ShapeDtypeStruct((8, d), jnp.float32)],
    )(h1s, aggs, wr, wn, cb, w2t, b2)


def _final_body(st_ref, g2_ref, be2_ref, y2_ref, x_ref, out_ref, *, n):
    mu = st_ref[0:1, :] / n
    ex2 = st_ref[1:2, :] / n
    var = ex2 - mu * mu
    a = g2_ref[...] * lax.rsqrt(var + _EPS)
    dd = be2_ref[...] - a * mu
    out_ref[...] = a * y2_ref[...] + dd + x_ref[...]


def _final(st, g2, be2, y2, x, rows_per_blk):
    n, d = x.shape
    grid = n // rows_per_blk
    body = functools.partial(_final_body, n=n)
    return pl.pallas_call(
        body,
        grid=(grid,),
        in_specs=[pl.BlockSpec((8, d), lambda i: (0, 0)),
                  pl.BlockSpec((1, d), lambda i: (0, 0)),
                  pl.BlockSpec((1, d), lambda i: (0, 0)),
                  pl.BlockSpec((rows_per_blk, d), lambda i: (i, 0)),
                  pl.BlockSpec((rows_per_blk, d), lambda i: (i, 0))],
        out_specs=pl.BlockSpec((rows_per_blk, d), lambda i: (i, 0)),
        out_shape=jax.ShapeDtypeStruct((n, d), jnp.float32),
    )(st, g2, be2, y2, x)


def kernel(x, edge_index, fc1_W, fc1_b, bn1_g, bn1_b, Wroot, Wnbr, conv_b,
           fc2_W, fc2_b, bn2_g, bn2_b):
    n, d = x.shape
    h = fc1_W.shape[0]
    e = edge_index.shape[1]
    rows_per_blk = 2000

    del fc1_b
    g, m = _xstats(x, rows_per_blk)
    w1t = fc1_W.T
    h1s = _h1(g, m, w1t, bn1_g.reshape(1, h), bn1_b.reshape(1, h), x,
              rows_per_blk)

    n_chunks = -(-e // (16 * 128 * 2 * _NBUF)) * 2 * _NBUF
    e_pad = n_chunks * 16 * 128
    nsp = -(-(n + 16) // 2048) * 2048
    pad = e_pad - e
    src = edge_index[0]
    dst = edge_index[1]
    fill = jnp.arange(pad, dtype=jnp.int32)
    src_p = jnp.concatenate([src, (fill * 97) % n])
    dst_p = jnp.concatenate([dst, n + fill % (nsp - n)])
    qoff = jnp.arange(_Q, dtype=jnp.int32)[:, None] * n
    src4 = (src_p[None, :] + qoff).reshape(_Q, 16, n_chunks, 128)
    dst2 = dst_p.reshape(16, n_chunks, 128)

    h1_flat = h1s.reshape(_Q * n, _QW)
    agg4 = _sc_agg(h1_flat, src4, dst2, n, nsp)
    aggs = agg4.reshape(_Q, n, _QW)

    wr = Wroot.T.reshape(_Q, _QW, 2 * h)
    wn = Wnbr.T.reshape(_Q, _QW, 2 * h)
    w2t = fc2_W.T
    y2, st = _h2(h1s, aggs, wr, wn, conv_b.reshape(1, 2 * h), w2t,
                 fc2_b.reshape(1, d), rows_per_blk)
    return _final(st, bn2_g.reshape(1, d), bn2_b.reshape(1, d), y2, x,
                  rows_per_blk)


# speedup vs baseline: 9.4228x; 1.3197x over previous
import functools

import jax
import jax.numpy as jnp
from jax import lax
from jax.experimental import pallas as pl
from jax.experimental.pallas import tpu as pltpu
from jax.experimental.pallas import tpu_sc as plsc

_EPS = 1e-5
_Q = 4
_QW = 64


def _xstats_body(x_ref, g_ref, m_ref):
    i = pl.program_id(0)
    xb = x_ref[...]
    gg = lax.dot_general(xb, xb, (((0,), (0,)), ((), ())),
                         preferred_element_type=jnp.float32)
    ms = jnp.sum(xb, axis=0, keepdims=True)

    @pl.when(i == 0)
    def _():
        g_ref[...] = gg
        m_ref[...] = jnp.zeros_like(m_ref)
        m_ref[0:1, :] = ms

    @pl.when(i > 0)
    def _():
        g_ref[...] += gg
        m_ref[0:1, :] += ms


def _xstats(x, rows_per_blk):
    n, d = x.shape
    grid = n // rows_per_blk
    return pl.pallas_call(
        _xstats_body,
        grid=(grid,),
        in_specs=[pl.BlockSpec((rows_per_blk, d), lambda i: (i, 0))],
        out_specs=[pl.BlockSpec((d, d), lambda i: (0, 0)),
                   pl.BlockSpec((8, d), lambda i: (0, 0))],
        out_shape=[jax.ShapeDtypeStruct((d, d), jnp.float32),
                   jax.ShapeDtypeStruct((8, d), jnp.float32)],
    )(x)


def _h1_body(g_ref, m_ref, w1t_ref, g1_ref, be1_ref, x_ref, out_ref, ac_ref,
             *, n):
    @pl.when(pl.program_id(0) == 0)
    def _():
        w1t = w1t_ref[...]
        t = jnp.dot(g_ref[...], w1t, preferred_element_type=jnp.float32)
        ex2 = jnp.sum(w1t * t, axis=0, keepdims=True) / n
        mu0 = jnp.dot(m_ref[0:1, :], w1t,
                      preferred_element_type=jnp.float32) / n
        var = ex2 - mu0 * mu0
        a = g1_ref[...] * lax.rsqrt(var + _EPS)
        ac_ref[0:1, :] = a
        ac_ref[1:2, :] = be1_ref[...] - a * mu0
    y = jnp.dot(x_ref[...], w1t_ref[...], preferred_element_type=jnp.float32)
    h = ac_ref[0:1, :] * y + ac_ref[1:2, :]
    for q in range(_Q):
        out_ref[q] = h[:, q * _QW:(q + 1) * _QW]


def _h1(g, m, w1t, g1, be1, x, rows_per_blk):
    n, d = x.shape
    h = w1t.shape[1]
    grid = n // rows_per_blk
    body = functools.partial(_h1_body, n=n)
    return pl.pallas_call(
        body,
        grid=(grid,),
        in_specs=[pl.BlockSpec((d, d), lambda i: (0, 0)),
                  pl.BlockSpec((8, d), lambda i: (0, 0)),
                  pl.BlockSpec((d, h), lambda i: (0, 0)),
                  pl.BlockSpec((1, h), lambda i: (0, 0)),
                  pl.BlockSpec((1, h), lambda i: (0, 0)),
                  pl.BlockSpec((rows_per_blk, d), lambda i: (i, 0))],
        out_specs=pl.BlockSpec((_Q, rows_per_blk, _QW), lambda i: (0, i, 0)),
        out_shape=jax.ShapeDtypeStruct((_Q, n, _QW), jnp.float32),
        scratch_shapes=[pltpu.VMEM((8, h), jnp.float32)],
    )(g, m, w1t, g1, be1, x)


_NBUF = 10
_CPG = 2


def _sc_agg_body(h1_ref, src_ref, dst_ref, out_ref,
                 isrc_v, idst_v, rows_v, acc_sh,
                 *sems,
                 **kw):
    n_chunks = kw["n_chunks"]
    rows_per_tile_out = kw["rows_per_tile_out"]
    zero_copies = kw["zero_copies"]
    n_groups = n_chunks // _NBUF
    gsems = sems[:_NBUF]
    sem_is = sems[_NBUF:_NBUF + 2]
    sem_id = sems[_NBUF + 2:_NBUF + 4]
    c = lax.axis_index("c")
    s = lax.axis_index("s")

    nb = _NBUF // _CPG

    def _start_idx(q, g, par):
        pltpu.async_copy(src_ref.at[q, s, pl.ds(g * nb, nb)],
                         isrc_v.at[par], sem_is[par])
        pltpu.async_copy(dst_ref.at[s, pl.ds(g * _NBUF, _NBUF)],
                         idst_v.at[par], sem_id[par])

    def _wait_idx(q, g, par):
        pltpu.make_async_copy(src_ref.at[q, s, pl.ds(g * nb, nb)],
                              isrc_v.at[par], sem_is[par]).wait()
        pltpu.make_async_copy(dst_ref.at[s, pl.ds(g * _NBUF, _NBUF)],
                              idst_v.at[par], sem_id[par]).wait()

    for p in range(2):
        q = 2 * c + p

        def _zrow(r, carry):
            for jj in range(_QW // 16):
                rows_v[0, r, pl.ds(jj * 16, 16)] = jnp.zeros((16,),
                                                             jnp.float32)
            return carry
        lax.fori_loop(0, 128, _zrow, 0)

        def _zcopy(k, carry):
            pltpu.sync_copy(rows_v.at[0, pl.ds(0, 128)],
                            acc_sh.at[pl.ds(s * (zero_copies * 128)
                                            + k * 128, 128)])
            return carry
        lax.fori_loop(0, zero_copies, _zcopy, 0)
        plsc.subcore_barrier()

        _start_idx(q, 0, 0)
        _start_idx(q, 1, 1)

        def _iter2(k, carry):
            for par in range(2):
                g = 2 * k + par
                _wait_idx(q, g, par)
                cps = [pltpu.async_copy(
                    h1_ref.at[isrc_v.at[par, j]],
                    rows_v.at[j], gsems[j]) for j in range(nb)]
                for j in range(nb):
                    cps[j].wait()
                    for hh in range(_CPG):
                        pltpu.sync_copy(
                            rows_v.at[j, pl.ds(hh * 128, 128)],
                            acc_sh.at[idst_v.at[par, j * _CPG + hh]],
                            add=True)

                @pl.when(g + 2 < n_groups)
                def _():
                    _start_idx(q, g + 2, par)
            return carry
        lax.fori_loop(0, n_groups // 2, _iter2, 0)
        plsc.subcore_barrier()

        pltpu.sync_copy(
            acc_sh.at[pl.ds(s * rows_per_tile_out, rows_per_tile_out)],
            out_ref.at[q, s])
        plsc.subcore_barrier()


def _sc_agg(h1_flat, src4, dst2, n, nsp):
    n_chunks = src4.shape[2]
    rows_per_tile_out = n // 16
    zero_copies = nsp // (16 * 128)
    body = functools.partial(_sc_agg_body, n_chunks=n_chunks,
                             rows_per_tile_out=rows_per_tile_out,
                             zero_copies=zero_copies)
    kern = pl.kernel(
        body,
        out_type=jax.ShapeDtypeStruct((_Q, 16, rows_per_tile_out, _QW),
                                      jnp.float32),
        mesh=plsc.VectorSubcoreMesh(core_axis_name="c", subcore_axis_name="s"),
        compiler_params=pltpu.CompilerParams(use_tc_tiling_on_sc=False),
        scratch_types=[
            pltpu.VMEM((2, _NBUF // _CPG, _CPG * 128), jnp.int32),
            pltpu.VMEM((2, _NBUF, 128), jnp.int32),
            pltpu.VMEM((_NBUF // _CPG, _CPG * 128, _QW), jnp.float32),
            pltpu.VMEM_SHARED((nsp, _QW), jnp.float32),
        ] + [pltpu.SemaphoreType.DMA] * (_NBUF + 4),
    )
    return kern(h1_flat, src4, dst2)


def _h2_body(h1_ref, agg_ref, wr_ref, wn_ref, cb_ref, w2t_ref, b2_ref,
             y2_ref, st_ref):
    i = pl.program_id(0)
    h2 = cb_ref[...]
    for q in range(_Q):
        h2 = (h2
              + jnp.dot(h1_ref[q], wr_ref[q],
                        preferred_element_type=jnp.float32)
              + jnp.dot(agg_ref[q], wn_ref[q],
                        preferred_element_type=jnp.float32))
    y2 = jnp.dot(h2, w2t_ref[...], preferred_element_type=jnp.float32) \
        + b2_ref[...]
    y2_ref[...] = y2
    s1 = jnp.sum(y2, axis=0, keepdims=True)
    s2 = jnp.sum(y2 * y2, axis=0, keepdims=True)

    @pl.when(i == 0)
    def _():
        st_ref[...] = jnp.zeros_like(st_ref)
        st_ref[0:1, :] = s1
        st_ref[1:2, :] = s2

    @pl.when(i > 0)
    def _():
        st_ref[0:1, :] += s1
        st_ref[1:2, :] += s2


def _h2(h1s, aggs, wr, wn, cb, w2t, b2, rows_per_blk):
    _, n, _ = h1s.shape
    h2dim = wr.shape[2]
    d = w2t.shape[1]
    grid = n // rows_per_blk
    return pl.pallas_call(
        _h2_body,
        grid=(grid,),
        in_specs=[pl.BlockSpec((_Q, rows_per_blk, _QW), lambda i: (0, i, 0)),
                  pl.BlockSpec((_Q, rows_per_blk, _QW), lambda i: (0, i, 0)),
                  pl.BlockSpec((_Q, _QW, h2dim), lambda i: (0, 0, 0)),
                  pl.BlockSpec((_Q, _QW, h2dim), lambda i: (0, 0, 0)),
                  pl.BlockSpec((1, h2dim), lambda i: (0, 0)),
                  pl.BlockSpec((h2dim, d), lambda i: (0, 0)),
                  pl.BlockSpec((1, d), lambda i: (0, 0))],
        out_specs=[pl.BlockSpec((rows_per_blk, d), lambda i: (i, 0)),
                   pl.BlockSpec((8, d), lambda i: (0, 0))],
        out_shape=[jax.ShapeDtypeStruct((n, d), jnp.float32),
                   jax.ShapeDtypeStruct((8, d), jnp.float32)],
    )(h1s, aggs, wr, wn, cb, w2t, b2)


def _final_body(st_ref, g2_ref, be2_ref, y2_ref, x_ref, out_ref, *, n):
    mu = st_ref[0:1, :] / n
    ex2 = st_ref[1:2, :] / n
    var = ex2 - mu * mu
    a = g2_ref[...] * lax.rsqrt(var + _EPS)
    dd = be2_ref[...] - a * mu
    out_ref[...] = a * y2_ref[...] + dd + x_ref[...]


def _final(st, g2, be2, y2, x, rows_per_blk):
    n, d = x.shape
    grid = n // rows_per_blk
    body = functools.partial(_final_body, n=n)
    return pl.pallas_call(
        body,
        grid=(grid,),
        in_specs=[pl.BlockSpec((8, d), lambda i: (0, 0)),
                  pl.BlockSpec((1, d), lambda i: (0, 0)),
                  pl.BlockSpec((1, d), lambda i: (0, 0)),
                  pl.BlockSpec((rows_per_blk, d), lambda i: (i, 0)),
                  pl.BlockSpec((rows_per_blk, d), lambda i: (i, 0))],
        out_specs=pl.BlockSpec((rows_per_blk, d), lambda i: (i, 0)),
        out_shape=jax.ShapeDtypeStruct((n, d), jnp.float32),
    )(st, g2, be2, y2, x)


def kernel(x, edge_index, fc1_W, fc1_b, bn1_g, bn1_b, Wroot, Wnbr, conv_b,
           fc2_W, fc2_b, bn2_g, bn2_b):
    n, d = x.shape
    h = fc1_W.shape[0]
    e = edge_index.shape[1]
    rows_per_blk = 2000

    del fc1_b
    g, m = _xstats(x, rows_per_blk)
    w1t = fc1_W.T
    h1s = _h1(g, m, w1t, bn1_g.reshape(1, h), bn1_b.reshape(1, h), x,
              rows_per_blk)

    n_chunks = -(-e // (16 * 128 * 2 * _NBUF)) * 2 * _NBUF
    e_pad = n_chunks * 16 * 128
    nsp = -(-(n + 16) // 2048) * 2048
    pad = e_pad - e
    src = edge_index[0]
    dst = edge_index[1]
    fill = jnp.arange(pad, dtype=jnp.int32)
    src_p = jnp.concatenate([src, (fill * 97) % n])
    dst_p = jnp.concatenate([dst, n + fill % (nsp - n)])
    qoff = jnp.arange(_Q, dtype=jnp.int32)[:, None] * n
    src4 = (src_p[None, :] + qoff).reshape(_Q, 16, n_chunks // _CPG,
                                           _CPG * 128)
    dst2 = dst_p.reshape(16, n_chunks, 128)

    h1_flat = h1s.reshape(_Q * n, _QW)
    agg4 = _sc_agg(h1_flat, src4, dst2, n, nsp)
    aggs = agg4.reshape(_Q, n, _QW)

    wr = Wroot.T.reshape(_Q, _QW, 2 * h)
    wn = Wnbr.T.reshape(_Q, _QW, 2 * h)
    w2t = fc2_W.T
    y2, st = _h2(h1s, aggs, wr, wn, conv_b.reshape(1, 2 * h), w2t,
                 fc2_b.reshape(1, d), rows_per_blk)
    return _final(st, bn2_g.reshape(1, d), bn2_b.reshape(1, d), y2, x,
                  rows_per_blk)
